# baseline (device time: 10527 ns/iter reference)
import jax
import jax.numpy as jnp
from jax import lax
from jax.experimental import pallas as pl
from jax.experimental.pallas import tpu as pltpu

M_HALF = 512
D = 512
NC = 4
CH = M_HALF // NC


def kernel(partial, gamma):
    partial2d = partial.reshape(2 * M_HALF, D)
    gamma2d = gamma.reshape(1, D)

    def body(x_hbm, g_hbm, out_hbm, xmine, xsend, gv, outv,
             sendq, recvq, sscale, rscale,
             mine_sems, send_stage_sems, out_sems, g_sem,
             qsend_sems, qrecv_sems, ssend_sems, srecv_sems):
        my_x = lax.axis_index("x")
        my_y = lax.axis_index("y")
        my_z = lax.axis_index("z")
        xpeer = (1 - my_x, my_y, my_z)

        src_base = (1 - my_x) * M_HALF
        my_base = my_x * M_HALF
        cp_g = pltpu.make_async_copy(g_hbm, gv, g_sem)
        cp_g.start()
        cp_send = []
        cp_mine = []
        for c in range(NC):
            sl = pl.ds(c * CH, CH)
            cp = pltpu.make_async_copy(
                x_hbm.at[pl.ds(src_base + c * CH, CH), :], xsend.at[sl],
                send_stage_sems.at[c])
            cp.start()
            cp_send.append(cp)
            cp = pltpu.make_async_copy(
                x_hbm.at[pl.ds(my_base + c * CH, CH), :], xmine.at[sl],
                mine_sems.at[c])
            cp.start()
            cp_mine.append(cp)

        barrier_sem = pltpu.get_barrier_semaphore()
        pl.semaphore_signal(
            barrier_sem, inc=1, device_id=xpeer,
            device_id_type=pl.DeviceIdType.MESH,
        )
        pl.semaphore_wait(barrier_sem, 1)

        data_rd = []
        scale_rd = []
        for c in range(NC):
            sl = pl.ds(c * CH, CH)
            tile = pl.ds(8 * c, 8)
            cp_send[c].wait()
            chunk = xsend[sl, :]
            absmax = jnp.maximum(jnp.max(jnp.abs(chunk)), 1e-20)
            sscale[tile, :] = jnp.full((8, 128), absmax / 127.0, jnp.float32)
            srd = pltpu.make_async_remote_copy(
                src_ref=sscale.at[tile], dst_ref=rscale.at[tile],
                send_sem=ssend_sems.at[c], recv_sem=srecv_sems.at[c],
                device_id=xpeer, device_id_type=pl.DeviceIdType.MESH,
            )
            srd.start()
            scale_rd.append(srd)
            sendq[sl, :] = jnp.rint(chunk * (127.0 / absmax)).astype(jnp.int8)
            rd = pltpu.make_async_remote_copy(
                src_ref=sendq.at[sl], dst_ref=recvq.at[sl],
                send_sem=qsend_sems.at[c], recv_sem=qrecv_sems.at[c],
                device_id=xpeer, device_id_type=pl.DeviceIdType.MESH,
            )
            rd.start()
            data_rd.append(rd)

        cp_g.wait()
        g = gv[:, :]

        cp_out = []
        for c in range(NC):
            sl = pl.ds(c * CH, CH)
            scale_rd[c].wait_recv()
            data_rd[c].wait_recv()
            cp_mine[c].wait()
            s = rscale[8 * c, 0]
            y = xmine[sl, :] + recvq[sl, :].astype(jnp.float32) * s
            rms = jnp.sqrt(jnp.mean(y * y, axis=-1, keepdims=True) + 1e-6)
            outv[sl, :] = y / rms * g
            cp = pltpu.make_async_copy(outv.at[sl], out_hbm.at[sl],
                                       out_sems.at[c])
            cp.start()
            cp_out.append(cp)

        for c in range(NC):
            cp_out[c].wait()
            data_rd[c].wait_send()
            scale_rd[c].wait_send()

    return pl.pallas_call(
        body,
        out_shape=jax.ShapeDtypeStruct((M_HALF, D), jnp.float32),
        in_specs=[
            pl.BlockSpec(memory_space=pl.ANY),
            pl.BlockSpec(memory_space=pl.ANY),
        ],
        out_specs=pl.BlockSpec(memory_space=pl.ANY),
        scratch_shapes=[
            pltpu.VMEM((M_HALF, D), jnp.float32),
            pltpu.VMEM((M_HALF, D), jnp.float32),
            pltpu.VMEM((1, D), jnp.float32),
            pltpu.VMEM((M_HALF, D), jnp.float32),
            pltpu.VMEM((M_HALF, D), jnp.int8),
            pltpu.VMEM((M_HALF, D), jnp.int8),
            pltpu.VMEM((8 * NC, 128), jnp.float32),
            pltpu.VMEM((8 * NC, 128), jnp.float32),
            pltpu.SemaphoreType.DMA((NC,)),
            pltpu.SemaphoreType.DMA((NC,)),
            pltpu.SemaphoreType.DMA((NC,)),
            pltpu.SemaphoreType.DMA,
            pltpu.SemaphoreType.DMA((NC,)),
            pltpu.SemaphoreType.DMA((NC,)),
            pltpu.SemaphoreType.DMA((NC,)),
            pltpu.SemaphoreType.DMA((NC,)),
        ],
        compiler_params=pltpu.CompilerParams(collective_id=0),
    )(partial2d, gamma2d)


# device time: 10034 ns/iter; 1.0491x vs baseline; 1.0491x over previous
import jax
import jax.numpy as jnp
from jax import lax
from jax.experimental import pallas as pl
from jax.experimental.pallas import tpu as pltpu

M_HALF = 512
D = 512
NC = 4
CH = M_HALF // NC


def kernel(partial, gamma):
    partial2d = partial.reshape(2 * M_HALF, D)
    gamma2d = gamma.reshape(1, D)

    def body(x_ref, g_ref, out_ref, send_ref, recv_ref,
             sscale_ref, rscale_ref, send_sems, recv_sems,
             scale_send_sem, scale_recv_sem):
        my_x = lax.axis_index("x")
        my_y = lax.axis_index("y")
        my_z = lax.axis_index("z")
        xpeer = (1 - my_x, my_y, my_z)

        barrier_sem = pltpu.get_barrier_semaphore()
        pl.semaphore_signal(
            barrier_sem, inc=1, device_id=xpeer,
            device_id_type=pl.DeviceIdType.MESH,
        )
        pl.semaphore_wait(barrier_sem, 1)

        src_base = (1 - my_x) * M_HALF
        block = x_ref[pl.ds(src_base, M_HALF), :]
        absmax = jnp.max(jnp.abs(block))
        scale = jnp.maximum(absmax, 1e-20) / 127.0
        sscale_ref[:, :] = jnp.full((8, 128), scale, jnp.float32)
        scale_rd = pltpu.make_async_remote_copy(
            src_ref=sscale_ref, dst_ref=rscale_ref,
            send_sem=scale_send_sem, recv_sem=scale_recv_sem,
            device_id=xpeer, device_id_type=pl.DeviceIdType.MESH,
        )
        scale_rd.start()

        inv = 127.0 / jnp.maximum(absmax, 1e-20)
        rdmas = []
        for c in range(NC):
            sl = pl.ds(c * CH, CH)
            chunk = x_ref[pl.ds(src_base + c * CH, CH), :]
            send_ref[sl, :] = jnp.rint(chunk * inv).astype(jnp.int8)
            rd = pltpu.make_async_remote_copy(
                src_ref=send_ref.at[sl],
                dst_ref=recv_ref.at[sl],
                send_sem=send_sems.at[c],
                recv_sem=recv_sems.at[c],
                device_id=xpeer,
                device_id_type=pl.DeviceIdType.MESH,
            )
            rd.start()
            rdmas.append(rd)

        scale_rd.wait_recv()
        peer_scale = rscale_ref[0, 0]
        g = g_ref[:, :]
        my_base = my_x * M_HALF
        for c in range(NC):
            rdmas[c].wait_recv()
            sl = pl.ds(c * CH, CH)
            remote = recv_ref[sl, :].astype(jnp.float32) * peer_scale
            y = x_ref[pl.ds(my_base + c * CH, CH), :] + remote
            rms = jnp.sqrt(jnp.mean(y * y, axis=-1, keepdims=True) + 1e-6)
            out_ref[sl, :] = y / rms * g

        scale_rd.wait_send()
        for c in range(NC):
            rdmas[c].wait_send()

    return pl.pallas_call(
        body,
        out_shape=jax.ShapeDtypeStruct((M_HALF, D), jnp.float32),
        in_specs=[
            pl.BlockSpec(memory_space=pltpu.VMEM),
            pl.BlockSpec(memory_space=pltpu.VMEM),
        ],
        out_specs=pl.BlockSpec(memory_space=pltpu.VMEM),
        scratch_shapes=[
            pltpu.VMEM((M_HALF, D), jnp.int8),
            pltpu.VMEM((M_HALF, D), jnp.int8),
            pltpu.VMEM((8, 128), jnp.float32),
            pltpu.VMEM((8, 128), jnp.float32),
            pltpu.SemaphoreType.DMA((NC,)),
            pltpu.SemaphoreType.DMA((NC,)),
            pltpu.SemaphoreType.DMA,
            pltpu.SemaphoreType.DMA,
        ],
        compiler_params=pltpu.CompilerParams(collective_id=0),
    )(partial2d, gamma2d)
